# baseline (speedup 1.0000x reference)
import functools

import jax
import jax.numpy as jnp
from jax import lax
from jax.experimental import pallas as pl
from jax.experimental.pallas import tpu as pltpu
from jax.experimental.pallas import tpu_sc as plsc

B = 16384
F = 26
V = 100000
D = 16
DENSE = 13
OUT_W = F * D + DENSE

NC = 2
NS = 16
NW = NC * NS
BPW = B // NW
CHUNK_B = 128
NCHUNK = BPW // CHUNK_B
ROWS = F * CHUNK_B
FLO = 13
HROWS = FLO * CHUNK_B

_mesh = plsc.VectorSubcoreMesh(core_axis_name="c", subcore_axis_name="s")


@functools.partial(
    pl.kernel,
    mesh=_mesh,
    out_type=jax.ShapeDtypeStruct((B, OUT_W), jnp.float32),
    scratch_types=[
        pltpu.VMEM((ROWS,), jnp.int32),
        pltpu.VMEM((ROWS,), jnp.int32),
        pltpu.VMEM((ROWS, D), jnp.float32),
        pltpu.VMEM((ROWS, D), jnp.float32),
        pltpu.VMEM((CHUNK_B, DENSE), jnp.float32),
        pltpu.VMEM((CHUNK_B, DENSE), jnp.float32),
        pltpu.SemaphoreType.DMA,
        pltpu.SemaphoreType.DMA,
    ],
    compiler_params=pltpu.CompilerParams(use_tc_tiling_on_sc=False),
)
def _encode(idx_hbm, table_lo_hbm, table_hi_hbm, dense_hbm, out_hbm, idx0,
            idx1, rows0, rows1, dense0, dense1, gsem, wsem):
    wid = lax.axis_index("s") * NC + lax.axis_index("c")
    idx_v = (idx0, idx1)
    rows_v = (rows0, rows1)
    dense_v = (dense0, dense1)

    def stage(c):
        buf = c % 2
        base_b = wid * BPW + c * CHUNK_B
        pltpu.sync_copy(idx_hbm.at[wid, c], idx_v[buf])
        pltpu.async_copy(
            table_lo_hbm.at[idx_v[buf].at[pl.ds(0, HROWS)]],
            rows_v[buf].at[pl.ds(0, HROWS)],
            gsem,
        )
        pltpu.async_copy(
            table_hi_hbm.at[idx_v[buf].at[pl.ds(HROWS, ROWS - HROWS)]],
            rows_v[buf].at[pl.ds(HROWS, ROWS - HROWS)],
            gsem,
        )
        pltpu.sync_copy(dense_hbm.at[pl.ds(base_b, CHUNK_B)], dense_v[buf])

    def emit(c):
        buf = c % 2
        base_b = wid * BPW + c * CHUNK_B
        pltpu.make_async_copy(
            table_lo_hbm.at[idx_v[buf].at[pl.ds(0, HROWS)]],
            rows_v[buf].at[pl.ds(0, HROWS)],
            gsem,
        ).wait()
        pltpu.make_async_copy(
            table_hi_hbm.at[idx_v[buf].at[pl.ds(HROWS, ROWS - HROWS)]],
            rows_v[buf].at[pl.ds(HROWS, ROWS - HROWS)],
            gsem,
        ).wait()

        @pl.loop(0, F)
        def _fire_write(f):
            pltpu.async_copy(
                rows_v[buf].at[pl.ds(f * CHUNK_B, CHUNK_B)],
                out_hbm.at[pl.ds(base_b, CHUNK_B), pl.ds(f * D, D)],
                wsem,
            )

        pltpu.async_copy(
            dense_v[buf],
            out_hbm.at[pl.ds(base_b, CHUNK_B), pl.ds(F * D, DENSE)],
            wsem,
        )

    def drain(c):
        buf = c % 2
        base_b = wid * BPW + c * CHUNK_B

        @pl.loop(0, F)
        def _wait_write(f):
            pltpu.make_async_copy(
                rows_v[buf].at[pl.ds(f * CHUNK_B, CHUNK_B)],
                out_hbm.at[pl.ds(base_b, CHUNK_B), pl.ds(f * D, D)],
                wsem,
            ).wait()

        pltpu.make_async_copy(
            dense_v[buf],
            out_hbm.at[pl.ds(base_b, CHUNK_B), pl.ds(F * D, DENSE)],
            wsem,
        ).wait()

    stage(0)
    for c in range(NCHUNK):
        if c + 1 < NCHUNK:
            if c - 1 >= 0:
                drain(c - 1)
            stage(c + 1)
        emit(c)
    drain(NCHUNK - 2)
    drain(NCHUNK - 1)


def kernel(sparse_indices, dense_x, tables):
    offs = (jnp.arange(F, dtype=jnp.int32) % FLO) * V
    flat_idx = sparse_indices + offs[None, :]
    flat_idx = (
        flat_idx.reshape(NW, NCHUNK, CHUNK_B, F)
        .transpose(0, 1, 3, 2)
        .reshape(NW, NCHUNK, ROWS)
    )
    table_lo = tables[:FLO].reshape(FLO * V, D)
    table_hi = tables[FLO:].reshape((F - FLO) * V, D)
    return _encode(flat_idx, table_lo, table_hi, dense_x.astype(jnp.float32))


# speedup vs baseline: 1.4865x; 1.4865x over previous
import functools

import jax
import jax.numpy as jnp
from jax import lax
from jax.experimental import pallas as pl
from jax.experimental.pallas import tpu as pltpu
from jax.experimental.pallas import tpu_sc as plsc

B = 16384
F = 26
V = 100000
D = 16
DENSE = 13
OUT_W = F * D + DENSE

NC = 2
NS = 16
NW = NC * NS
BPW = B // NW
CHUNK_B = 128
NCHUNK = BPW // CHUNK_B
ROWS = F * CHUNK_B

_mesh = plsc.VectorSubcoreMesh(core_axis_name="c", subcore_axis_name="s")


@functools.partial(
    pl.kernel,
    mesh=_mesh,
    out_type=jax.ShapeDtypeStruct((B, OUT_W), jnp.float32),
    scratch_types=[
        pltpu.VMEM((ROWS,), jnp.int32),
        pltpu.VMEM((ROWS,), jnp.int32),
        pltpu.VMEM((ROWS, D), jnp.float32),
        pltpu.VMEM((ROWS, D), jnp.float32),
        pltpu.VMEM((CHUNK_B, DENSE), jnp.float32),
        pltpu.VMEM((CHUNK_B, DENSE), jnp.float32),
        pltpu.SemaphoreType.DMA,
        pltpu.SemaphoreType.DMA,
    ],
    compiler_params=pltpu.CompilerParams(use_tc_tiling_on_sc=False),
)
def _encode(idx_hbm, table_hbm, dense_hbm, out_hbm, idx0, idx1, rows0, rows1,
            dense0, dense1, gsem, wsem):
    wid = lax.axis_index("s") * NC + lax.axis_index("c")
    idx_v = (idx0, idx1)
    rows_v = (rows0, rows1)
    dense_v = (dense0, dense1)

    def stage(c):
        buf = c % 2
        base_b = wid * BPW + c * CHUNK_B
        pltpu.sync_copy(idx_hbm.at[wid, c], idx_v[buf])
        pltpu.async_copy(table_hbm.at[idx_v[buf]], rows_v[buf], gsem)
        pltpu.sync_copy(dense_hbm.at[pl.ds(base_b, CHUNK_B)], dense_v[buf])

    def emit(c):
        buf = c % 2
        base_b = wid * BPW + c * CHUNK_B
        pltpu.make_async_copy(
            table_hbm.at[idx_v[buf]], rows_v[buf], gsem
        ).wait()

        @pl.loop(0, F)
        def _fire_write(f):
            pltpu.async_copy(
                rows_v[buf].at[pl.ds(f * CHUNK_B, CHUNK_B)],
                out_hbm.at[pl.ds(base_b, CHUNK_B), pl.ds(f * D, D)],
                wsem,
            )

        pltpu.async_copy(
            dense_v[buf],
            out_hbm.at[pl.ds(base_b, CHUNK_B), pl.ds(F * D, DENSE)],
            wsem,
        )

    def drain(c):
        buf = c % 2
        base_b = wid * BPW + c * CHUNK_B

        @pl.loop(0, F)
        def _wait_write(f):
            pltpu.make_async_copy(
                rows_v[buf].at[pl.ds(f * CHUNK_B, CHUNK_B)],
                out_hbm.at[pl.ds(base_b, CHUNK_B), pl.ds(f * D, D)],
                wsem,
            ).wait()

        pltpu.make_async_copy(
            dense_v[buf],
            out_hbm.at[pl.ds(base_b, CHUNK_B), pl.ds(F * D, DENSE)],
            wsem,
        ).wait()

    stage(0)
    for c in range(NCHUNK):
        if c + 1 < NCHUNK:
            if c - 1 >= 0:
                drain(c - 1)
            stage(c + 1)
        emit(c)
    drain(NCHUNK - 2)
    drain(NCHUNK - 1)


def kernel(sparse_indices, dense_x, tables):
    flat_idx = sparse_indices + (jnp.arange(F, dtype=jnp.int32) * V)[None, :]
    flat_idx = (
        flat_idx.reshape(NW, NCHUNK, CHUNK_B, F)
        .transpose(0, 1, 3, 2)
        .reshape(NW, NCHUNK, ROWS)
    )
    table2d = tables.reshape(F * V, D)
    return _encode(flat_idx, table2d, dense_x.astype(jnp.float32))
